# Initial kernel scaffold; baseline (speedup 1.0000x reference)
import jax
import jax.numpy as jnp
from jax.experimental import pallas as pl


def kernel(question_seq, correctness_seq, mask_seq, question_neighbors, concept_neighbors, q2c, Wq, Wc, Wcorr, gru1_Wih, gru1_Whh, gru1_bih, gru1_bhh, gru2_Wih, gru2_Whh, gru2_bih, gru2_bhh, agg_W, agg_b, agg_last_W, agg_last_b, q_W, q_b, k_W, k_b, w_W, w_b, h1_init, h2_init):
    raise NotImplementedError("write your pallas kernel here")


# speedup vs baseline: 8.4085x; 8.4085x over previous
import functools
import jax
import jax.numpy as jnp
from jax import lax
from jax.experimental import pallas as pl
from jax.experimental.pallas import tpu as pltpu
from jax.experimental.pallas import tpu_sc as plsc

NQ, NC, D = 50000, 2000, 128
B, S = 64, 100
QN, CN, NCPQ, RK = 4, 10, 4, 10
NW = 32


def _lazy(builder):
    box = []

    def f(*a):
        if not box:
            box.append(builder())
        return box[0](*a)

    return f


def _make_sc_gather(V, Drow, dtype, Npad, group, chunk):
    pw = Npad // NW
    assert Npad % NW == 0 and pw % group == 0 and group % chunk == 0
    assert chunk <= 128 and chunk % 8 == 0 and pw % 8 == 0

    @functools.partial(
        pl.kernel,
        mesh=plsc.VectorSubcoreMesh(core_axis_name="c", subcore_axis_name="s"),
        out_type=jax.ShapeDtypeStruct((Npad, Drow), dtype),
        scratch_types=[
            pltpu.VMEM((pw,), jnp.int32),
            pltpu.VMEM((group, Drow), dtype),
            pltpu.SemaphoreType.DMA,
        ],
    )
    def k(table_hbm, idx_hbm, out_hbm, idx_v, rows_v, sem):
        wid = lax.axis_index("s") * 2 + lax.axis_index("c")
        base = wid * pw
        pltpu.sync_copy(idx_hbm.at[pl.ds(base, pw)], idx_v)
        for g in range(pw // group):
            cps = [
                pltpu.async_copy(
                    table_hbm.at[idx_v.at[pl.ds(g * group + j * chunk, chunk)]],
                    rows_v.at[pl.ds(j * chunk, chunk)], sem)
                for j in range(group // chunk)
            ]
            for c in cps:
                c.wait()
            pltpu.sync_copy(rows_v, out_hbm.at[pl.ds(base + g * group, group)])

    return k


def _build_sc_stage1():
    @functools.partial(
        pl.kernel,
        mesh=plsc.VectorSubcoreMesh(core_axis_name="c", subcore_axis_name="s"),
        out_type=[
            jax.ShapeDtypeStruct((B * S, D), jnp.float32),
            jax.ShapeDtypeStruct((B * S, D), jnp.int32),
            jax.ShapeDtypeStruct((NC * CN + 480, D), jnp.float32),
        ],
        scratch_types=[
            pltpu.VMEM((200,), jnp.int32),
            pltpu.VMEM((200, D), jnp.float32),
            pltpu.VMEM((200, D), jnp.int32),
            pltpu.VMEM((640,), jnp.int32),
            pltpu.VMEM((320, D), jnp.float32),
            pltpu.SemaphoreType.DMA,
            pltpu.SemaphoreType.DMA,
        ],
    )
    def k(wq_hbm, qnc_hbm, qidx_hbm, cidx_hbm,
          e_hbm, qncrows_hbm, cnrows_hbm,
          qi_v, erows_v, qrows_v, ci_v, crows_v, sem, sem2):
        wid = lax.axis_index("s") * 2 + lax.axis_index("c")
        qb = wid * 200
        pltpu.sync_copy(qidx_hbm.at[pl.ds(qb, 200)], qi_v)
        cps = [pltpu.async_copy(wq_hbm.at[qi_v.at[pl.ds(0, 104)]],
                                erows_v.at[pl.ds(0, 104)], sem),
               pltpu.async_copy(wq_hbm.at[qi_v.at[pl.ds(104, 96)]],
                                erows_v.at[pl.ds(104, 96)], sem),
               pltpu.async_copy(qnc_hbm.at[qi_v.at[pl.ds(0, 104)]],
                                qrows_v.at[pl.ds(0, 104)], sem),
               pltpu.async_copy(qnc_hbm.at[qi_v.at[pl.ds(104, 96)]],
                                qrows_v.at[pl.ds(104, 96)], sem)]
        cb = wid * 640
        pltpu.sync_copy(cidx_hbm.at[pl.ds(cb, 640)], ci_v)
        for g in range(2):
            cg = [pltpu.async_copy(
                wq_hbm.at[ci_v.at[pl.ds(g * 320 + off, n)]],
                crows_v.at[pl.ds(off, n)], sem2)
                for off, n in ((0, 128), (128, 128), (256, 64))]
            for c in cg:
                c.wait()
            pltpu.sync_copy(crows_v, cnrows_hbm.at[pl.ds(cb + g * 320, 320)])
        for c in cps:
            c.wait()
        pltpu.sync_copy(erows_v, e_hbm.at[pl.ds(qb, 200)])
        pltpu.sync_copy(qrows_v, qncrows_hbm.at[pl.ds(qb, 200)])

    return k


_sc_stage1 = _lazy(_build_sc_stage1)
_sc_gather_wc = _lazy(lambda: _make_sc_gather(NC, D, jnp.float32, 25600, 800, 80))
_sc_gather_tmean = _lazy(lambda: _make_sc_gather(NC, 2 * D, jnp.float32, 25600, 400, 80))
_sc_gather_state = _lazy(lambda: _make_sc_gather((S - 1) * B, D, jnp.float32, 64512, 672, 96))


def _b1_body(cn_ref, wc_ref, w1t_ref, b1_ref, out_ref):
    m = jnp.mean(cn_ref[...], axis=1)
    g1 = jnp.tanh(jnp.dot(m + wc_ref[...], w1t_ref[...],
                          preferred_element_type=jnp.float32) + b1_ref[...])
    out_ref[:, :D] = wc_ref[...]
    out_ref[:, D:] = g1


def _tc_tables(cnrows3, Wc, aggW1T, aggb1):
    nb = 5
    blk = NC // nb
    return pl.pallas_call(
        _b1_body,
        grid=(nb,),
        in_specs=[pl.BlockSpec((blk, CN, D), lambda i: (i, 0, 0)),
                  pl.BlockSpec((blk, D), lambda i: (i, 0)),
                  pl.BlockSpec((D, D), lambda i: (0, 0)),
                  pl.BlockSpec((1, D), lambda i: (0, 0))],
        out_specs=pl.BlockSpec((blk, 2 * D), lambda i: (i, 0)),
        out_shape=jax.ShapeDtypeStruct((NC, 2 * D), jnp.float32),
        interpret=False,
    )(cnrows3, Wc, aggW1T, aggb1)


def _proj_body(x_ref, wt_ref, b_ref, w1_ref, qm_ref, gq_ref):
    x = x_ref[...]
    qm_ref[...] = jnp.dot(x, wt_ref[...],
                          preferred_element_type=jnp.float32) + b_ref[...]
    gq_ref[...] = jnp.dot(x, w1_ref[...], preferred_element_type=jnp.float32)


def _tc_proj(rows, WT, b, w1tile):
    n = rows.shape[0]
    blk = 2048
    nb = (n + blk - 1) // blk
    return pl.pallas_call(
        _proj_body,
        grid=(nb,),
        in_specs=[pl.BlockSpec((blk, D), lambda i: (i, 0)),
                  pl.BlockSpec((D, D), lambda i: (0, 0)),
                  pl.BlockSpec((1, D), lambda i: (0, 0)),
                  pl.BlockSpec((D, 8), lambda i: (0, 0))],
        out_specs=[pl.BlockSpec((blk, D), lambda i: (i, 0)),
                   pl.BlockSpec((blk, 8), lambda i: (i, 0))],
        out_shape=[jax.ShapeDtypeStruct((n, D), jnp.float32),
                   jax.ShapeDtypeStruct((n, 8), jnp.float32)],
        interpret=False,
    )(rows, WT, b, w1tile)


def _topk_body(e_ref, sel_ref):
    e = e_ref[0]
    sc = lax.dot_general(e, e, (((1,), (1,)), ((), ())),
                         preferred_element_type=jnp.float32)
    s = sc[1:, :]
    tcol = lax.broadcasted_iota(jnp.int32, (S - 1, S), 0)
    jcol = lax.broadcasted_iota(jnp.int32, (S - 1, S), 1)
    s = jnp.where(jcol < tcol, s, -1e30)
    sel_ref[0] = jnp.zeros((S, 16), jnp.int32)
    sels = []
    for _ in range(RK):
        m = jnp.max(s, axis=1, keepdims=True)
        idx = jnp.min(jnp.where(s >= m, jcol, S + 1), axis=1, keepdims=True)
        sels.append(idx)
        s = jnp.where(jcol == idx, -1e30, s)
    sel_ref[0, 0:S - 1, 0:RK] = jnp.concatenate(sels, axis=1)


def _tc_topk(Eb):
    return pl.pallas_call(
        _topk_body,
        grid=(B,),
        in_specs=[pl.BlockSpec((1, S, D), lambda i: (i, 0, 0))],
        out_specs=pl.BlockSpec((1, S, 16), lambda i: (i, 0, 0)),
        out_shape=jax.ShapeDtypeStruct((B, S, 16), jnp.int32),
        interpret=False,
    )(Eb)


def _d_body(tr_ref, e_ref, mc_ref, wcorr_ref, w0t_ref, b0_ref,
            wlt_ref, bl_ref, wih1t_ref, bih1_ref, gi_ref):
    sm = jnp.mean(tr_ref[...], axis=1)
    e = e_ref[...]
    f0 = jnp.tanh(jnp.dot(sm[:, :D] + e, w0t_ref[...],
                          preferred_element_type=jnp.float32) + b0_ref[...])
    f0 = jnp.tanh(jnp.dot(sm[:, D:] + f0, w0t_ref[...],
                          preferred_element_type=jnp.float32) + b0_ref[...])
    embq = jnp.tanh(jnp.dot(f0, wlt_ref[...],
                            preferred_element_type=jnp.float32) + bl_ref[...])
    mask = mc_ref[:, 0:1]
    embq = jnp.where(mask > 0.5, embq, e)
    corr = mc_ref[:, 1:2]
    embr = jnp.where(corr > 0.5, wcorr_ref[1:2, :], wcorr_ref[0:1, :])
    x = jnp.concatenate([embq, embr], axis=1)
    gi_ref[...] = jnp.dot(x, wih1t_ref[...],
                          preferred_element_type=jnp.float32) + bih1_ref[...]


def _tc_dense(Trows3, E, mc, Wcorr, W0T, b0, WLT, bl, Wih1T, bih1):
    blk = 640
    nb = (B * S) // blk
    return pl.pallas_call(
        _d_body,
        grid=(nb,),
        in_specs=[pl.BlockSpec((blk, QN, 2 * D), lambda i: (i, 0, 0)),
                  pl.BlockSpec((blk, D), lambda i: (i, 0)),
                  pl.BlockSpec((blk, 8), lambda i: (i, 0)),
                  pl.BlockSpec((2, D), lambda i: (0, 0)),
                  pl.BlockSpec((D, D), lambda i: (0, 0)),
                  pl.BlockSpec((1, D), lambda i: (0, 0)),
                  pl.BlockSpec((D, D), lambda i: (0, 0)),
                  pl.BlockSpec((1, D), lambda i: (0, 0)),
                  pl.BlockSpec((2 * D, 3 * D), lambda i: (0, 0)),
                  pl.BlockSpec((1, 3 * D), lambda i: (0, 0))],
        out_specs=pl.BlockSpec((blk, 3 * D), lambda i: (i, 0)),
        out_shape=jax.ShapeDtypeStruct((B * S, 3 * D), jnp.float32),
        interpret=False,
    )(Trows3, E, mc, Wcorr, W0T, b0, WLT, bl, Wih1T, bih1)


def _gates(gi, gh, h):
    i_r, i_z, i_n = gi[:, :D], gi[:, D:2 * D], gi[:, 2 * D:]
    h_r, h_z, h_n = gh[:, :D], gh[:, D:2 * D], gh[:, 2 * D:]
    r = jax.nn.sigmoid(i_r + h_r)
    z = jax.nn.sigmoid(i_z + h_z)
    n = jnp.tanh(i_n + r * h_n)
    return (1.0 - z) * n + z * h


def _seq_body(gi1_ref, h1i_ref, h2i_ref, whh1t_ref, bhh1_ref,
              wih2t_ref, bih2_ref, whh2t_ref, bhh2_ref, st_ref):
    whh1t = whh1t_ref[...]
    wih2t = wih2t_ref[...]
    whh2t = whh2t_ref[...]
    bhh1 = bhh1_ref[...]
    bih2 = bih2_ref[...]
    bhh2 = bhh2_ref[...]

    def step(t, carry):
        h1, h2 = carry
        gi = gi1_ref[pl.ds(t, 1)].reshape(B, 3 * D)
        gh1 = jnp.dot(h1, whh1t, preferred_element_type=jnp.float32) + bhh1
        h1n = _gates(gi, gh1, h1)
        gi2 = jnp.dot(h1n, wih2t, preferred_element_type=jnp.float32) + bih2
        gh2 = jnp.dot(h2, whh2t, preferred_element_type=jnp.float32) + bhh2
        out = _gates(gi2, gh2, h2)
        st_ref[pl.ds(t, 1)] = out.reshape(1, B, D)
        h2n = jnp.where(t == 0, h2, out)
        return h1n, h2n

    lax.fori_loop(0, S - 1, step, (h1i_ref[...], h2i_ref[...]))


def _tc_seq(gi1_t, h1i, h2i, Whh1T, bhh1, Wih2T, bih2, Whh2T, bhh2):
    return pl.pallas_call(
        _seq_body,
        out_shape=jax.ShapeDtypeStruct((S - 1, B, D), jnp.float32),
        interpret=False,
    )(gi1_t, h1i, h2i, Whh1T, bhh1, Wih2T, bih2, Whh2T, bhh2)


def _att_body(qm_ref, ss_ref, kc_ref, gq_ref, gw_ref, mk_ref,
              kwt_ref, kb_ref, w2r_ref, out_ref):
    qm = qm_ref[...]
    kc = kc_ref[...]
    l0 = jnp.sum(qm * kc[:, None, :], axis=-1)
    ss = ss_ref[...]
    ks = lax.dot_general(ss, kwt_ref[...], (((2,), (0,)), ((), ())),
                         preferred_element_type=jnp.float32)
    ks = ks + kb_ref[...][None, :, :]
    lk = jnp.sum(qm[:, :, None, :] * ks[:, None, :, :], axis=-1)
    logits = jnp.concatenate([l0[:, :, None], lk], axis=2)
    mk = mk_ref[:, :RK + 1][:, None, :]
    lm = jnp.where(mk > 0.5, logits, -1e30)
    mx = jnp.max(lm, axis=(1, 2), keepdims=True)
    ex = jnp.exp(lm - mx)
    alpha = ex / jnp.sum(ex, axis=(1, 2), keepdims=True)
    ghs = jnp.sum(ss * w2r_ref[...][:, None, :], axis=-1)
    gh = jnp.concatenate([gw_ref[:, 0:1], ghs], axis=1)
    g = jax.nn.sigmoid(gq_ref[:, :NCPQ + 1][:, :, None] +
                       gh[:, None, :])
    pred = jnp.sum(jnp.where(mk > 0.5, alpha * g, 0.0), axis=(1, 2))
    out_ref[...] = jnp.broadcast_to(pred[:, None], pred.shape + (8,))


def _tc_att(Qm, Ssel, Kc, gq, gw8, maskf, kWT, kb, w2r):
    n = (S - 1) * B
    blk = 1056
    nb = n // blk
    return pl.pallas_call(
        _att_body,
        grid=(nb,),
        in_specs=[pl.BlockSpec((blk, NCPQ + 1, D), lambda i: (i, 0, 0)),
                  pl.BlockSpec((blk, RK, D), lambda i: (i, 0, 0)),
                  pl.BlockSpec((blk, D), lambda i: (i, 0)),
                  pl.BlockSpec((blk, 8), lambda i: (i, 0)),
                  pl.BlockSpec((blk, 8), lambda i: (i, 0)),
                  pl.BlockSpec((blk, 16), lambda i: (i, 0)),
                  pl.BlockSpec((D, D), lambda i: (0, 0)),
                  pl.BlockSpec((1, D), lambda i: (0, 0)),
                  pl.BlockSpec((1, D), lambda i: (0, 0))],
        out_specs=pl.BlockSpec((blk, 8), lambda i: (i, 0)),
        out_shape=jax.ShapeDtypeStruct((n, 8), jnp.float32),
        interpret=False,
    )(Qm, Ssel, Kc, gq, gw8, maskf, kWT, kb, w2r)


def kernel(question_seq, correctness_seq, mask_seq, question_neighbors,
           concept_neighbors, q2c, Wq, Wc, Wcorr,
           gru1_Wih, gru1_Whh, gru1_bih, gru1_bhh,
           gru2_Wih, gru2_Whh, gru2_bih, gru2_bhh,
           agg_W, agg_b, agg_last_W, agg_last_b,
           q_W, q_b, k_W, k_b, w_W, w_b, h1_init, h2_init):
    f32 = jnp.float32
    qflat = question_seq.T.reshape(-1)
    qnc128 = jnp.concatenate(
        [question_neighbors, q2c,
         jnp.zeros((NQ, D - 2 * QN), jnp.int32)], axis=1)
    cn_pad = jnp.concatenate(
        [concept_neighbors.reshape(-1),
         jnp.zeros((480,), jnp.int32)])

    E, qncrows, cnrows = _sc_stage1(Wq, qnc128, qflat, cn_pad)
    n1flat = qncrows[:, :QN].reshape(-1)
    c4flat = qncrows[:, QN:2 * QN].reshape(-1)

    Wc4 = _sc_gather_wc(Wc, c4flat)

    w1 = w_W[:, :D].T
    w2 = w_W[:, D:].T
    w1tile = jnp.broadcast_to(w1, (D, 8))
    w2tile = jnp.broadcast_to(w2, (D, 8))

    Tmean = _tc_tables(cnrows[:NC * CN].reshape(NC, CN, D), Wc,
                       agg_W[1].T, agg_b[1].reshape(1, D))
    Qm_c, gq_c8 = _tc_proj(Wc4, q_W.T, q_b.reshape(1, D), w1tile)
    Qm_q, gq_q8 = _tc_proj(E, q_W.T, q_b.reshape(1, D), w1tile)

    sel16 = _tc_topk(E.reshape(S, B, D).transpose(1, 0, 2))
    sel = sel16[:, :S - 1, :RK]

    Trows = _sc_gather_tmean(Tmean, n1flat)

    mc = jnp.zeros((B * S, 8), f32)
    mc = mc.at[:, 0].set((mask_seq.T.reshape(-1) != 0).astype(f32))
    mc = mc.at[:, 1].set(correctness_seq.T.reshape(-1).astype(f32))

    gi1 = _tc_dense(Trows.reshape(B * S, QN, 2 * D), E, mc, Wcorr,
                    agg_W[0].T, agg_b[0].reshape(1, D),
                    agg_last_W.T, agg_last_b.reshape(1, D),
                    gru1_Wih.T, gru1_bih.reshape(1, 3 * D))

    states = _tc_seq(gi1.reshape(S, B, 3 * D)[:S - 1], h1_init, h2_init,
                     gru1_Whh.T, gru1_bhh.reshape(1, 3 * D),
                     gru2_Wih.T, gru2_bih.reshape(1, 3 * D),
                     gru2_Whh.T, gru2_bhh.reshape(1, 3 * D))

    srows = states.reshape((S - 1) * B, D)
    K_rows, gw8 = _tc_proj(srows, k_W.T, k_b.reshape(1, D), w2tile)

    statetab = jnp.concatenate(
        [jnp.zeros((B, D), f32), srows[B:]], axis=0)

    bcol = jnp.arange(B, dtype=jnp.int32)[:, None, None]
    fid = (sel * B + bcol).transpose(1, 0, 2).reshape(-1)
    fid = jnp.concatenate([fid, jnp.zeros((64512 - fid.shape[0],), jnp.int32)])
    Ssel = _sc_gather_state(statetab, fid)[: (S - 1) * B * RK]
    Ssel = Ssel.reshape((S - 1) * B, RK, D)

    tarr = jnp.arange(S - 1)
    hv = (jnp.arange(RK + 1)[None, :] <= jnp.minimum(tarr, RK)[:, None])
    maskf = jnp.zeros((S - 1, 16), f32).at[:, :RK + 1].set(hv.astype(f32))
    maskf = jnp.broadcast_to(maskf[:, None, :], (S - 1, B, 16)).reshape(-1, 16)

    Qm = jnp.concatenate(
        [Qm_q.reshape(S, B, 1, D)[1:],
         Qm_c.reshape(S, B, NCPQ, D)[1:]], axis=2).reshape(-1, NCPQ + 1, D)
    gq = jnp.concatenate(
        [gq_q8.reshape(S, B, 8)[1:, :, :1],
         gq_c8.reshape(S, B, NCPQ, 8)[1:, :, :, 0]], axis=2).reshape(-1, NCPQ + 1)
    gq = gq + w_b[0]
    gq = jnp.concatenate(
        [gq, jnp.zeros(((S - 1) * B, 8 - NCPQ - 1), f32)], axis=1)

    pred8 = _tc_att(Qm, Ssel, K_rows, gq, gw8, maskf,
                    k_W.T, k_b.reshape(1, D), w2.reshape(1, D))
    pred = pred8[:, 0].reshape(S - 1, B).T

    y = jnp.concatenate(
        [pred[:, :1], jnp.zeros((B, 1), f32), pred[:, 1:]], axis=1)
    return y


# speedup vs baseline: 8.9779x; 1.0677x over previous
import functools
import jax
import jax.numpy as jnp
from jax import lax
from jax.experimental import pallas as pl
from jax.experimental.pallas import tpu as pltpu
from jax.experimental.pallas import tpu_sc as plsc

NQ, NC, D = 50000, 2000, 128
B, S = 64, 100
QN, CN, NCPQ, RK = 4, 10, 4, 10
NW = 32


def _lazy(builder):
    box = []

    def f(*a):
        if not box:
            box.append(builder())
        return box[0](*a)

    return f


def _make_sc_gather(V, Drow, dtype, Npad, group, chunk):
    pw = Npad // NW
    ng = pw // group
    assert Npad % NW == 0 and pw % group == 0 and group % chunk == 0
    assert chunk <= 128 and chunk % 8 == 0 and pw % 8 == 0

    @functools.partial(
        pl.kernel,
        mesh=plsc.VectorSubcoreMesh(core_axis_name="c", subcore_axis_name="s"),
        out_type=jax.ShapeDtypeStruct((Npad, Drow), dtype),
        scratch_types=[
            pltpu.VMEM((pw,), jnp.int32),
            pltpu.VMEM((group, Drow), dtype),
            pltpu.VMEM((group, Drow), dtype),
            pltpu.SemaphoreType.DMA,
            pltpu.SemaphoreType.DMA,
        ],
    )
    def k(table_hbm, idx_hbm, out_hbm, idx_v, rows0, rows1, semA, semB):
        wid = lax.axis_index("s") * 2 + lax.axis_index("c")
        base = wid * pw
        pltpu.sync_copy(idx_hbm.at[pl.ds(base, pw)], idx_v)
        bufs = (rows0, rows1)
        sems = (semA, semB)

        def fire(g):
            return [
                pltpu.async_copy(
                    table_hbm.at[idx_v.at[pl.ds(g * group + j * chunk, chunk)]],
                    bufs[g % 2].at[pl.ds(j * chunk, chunk)], sems[g % 2])
                for j in range(group // chunk)
            ]

        pend = fire(0)
        for g in range(ng):
            nxt = fire(g + 1) if g + 1 < ng else []
            for c in pend:
                c.wait()
            pltpu.sync_copy(bufs[g % 2], out_hbm.at[pl.ds(base + g * group, group)])
            pend = nxt

    return k


def _build_sc_stage1():
    @functools.partial(
        pl.kernel,
        mesh=plsc.VectorSubcoreMesh(core_axis_name="c", subcore_axis_name="s"),
        out_type=[
            jax.ShapeDtypeStruct((B * S, D), jnp.float32),
            jax.ShapeDtypeStruct((B * S, D), jnp.int32),
            jax.ShapeDtypeStruct((NC * CN + 480, D), jnp.float32),
        ],
        scratch_types=[
            pltpu.VMEM((200,), jnp.int32),
            pltpu.VMEM((200,), jnp.int32),
            pltpu.VMEM((200, D), jnp.float32),
            pltpu.VMEM((200, D), jnp.int32),
            pltpu.VMEM((640,), jnp.int32),
            pltpu.VMEM((320, D), jnp.float32),
            pltpu.SemaphoreType.DMA,
            pltpu.SemaphoreType.DMA,
        ],
    )
    def k(wq_hbm, qnc_hbm, qidx_hbm, qridx_hbm, cidx_hbm,
          e_hbm, qncrows_hbm, cnrows_hbm,
          qi_v, qri_v, erows_v, qrows_v, ci_v, crows_v, sem, sem2):
        wid = lax.axis_index("s") * 2 + lax.axis_index("c")
        qb = wid * 200
        pltpu.sync_copy(qidx_hbm.at[pl.ds(qb, 200)], qi_v)
        pltpu.sync_copy(qridx_hbm.at[pl.ds(qb, 200)], qri_v)
        cps = [pltpu.async_copy(wq_hbm.at[qi_v.at[pl.ds(0, 104)]],
                                erows_v.at[pl.ds(0, 104)], sem),
               pltpu.async_copy(wq_hbm.at[qi_v.at[pl.ds(104, 96)]],
                                erows_v.at[pl.ds(104, 96)], sem),
               pltpu.async_copy(qnc_hbm.at[qri_v.at[pl.ds(0, 104)]],
                                qrows_v.at[pl.ds(0, 104)], sem),
               pltpu.async_copy(qnc_hbm.at[qri_v.at[pl.ds(104, 96)]],
                                qrows_v.at[pl.ds(104, 96)], sem)]
        cb = wid * 640
        pltpu.sync_copy(cidx_hbm.at[pl.ds(cb, 640)], ci_v)
        for g in range(2):
            cg = [pltpu.async_copy(
                wq_hbm.at[ci_v.at[pl.ds(g * 320 + off, n)]],
                crows_v.at[pl.ds(off, n)], sem2)
                for off, n in ((0, 128), (128, 128), (256, 64))]
            for c in cg:
                c.wait()
            pltpu.sync_copy(crows_v, cnrows_hbm.at[pl.ds(cb + g * 320, 320)])
        for c in cps:
            c.wait()
        pltpu.sync_copy(erows_v, e_hbm.at[pl.ds(qb, 200)])
        pltpu.sync_copy(qrows_v, qncrows_hbm.at[pl.ds(qb, 200)])

    return k


_sc_stage1 = _lazy(_build_sc_stage1)
_sc_gather_wc = _lazy(lambda: _make_sc_gather(NC, D, jnp.float32, 25600, 400, 80))
_sc_gather_tmean = _lazy(lambda: _make_sc_gather(NC, 2 * D, jnp.float32, 25600, 200, 40))
_sc_gather_state = _lazy(lambda: _make_sc_gather((S - 1) * B, D, jnp.float32, 64000, 400, 80))


def _b1_body(cn_ref, wc_ref, w1t_ref, b1_ref, out_ref):
    m = jnp.mean(cn_ref[...], axis=1)
    g1 = jnp.tanh(jnp.dot(m + wc_ref[...], w1t_ref[...],
                          preferred_element_type=jnp.float32) + b1_ref[...])
    out_ref[:, :D] = wc_ref[...]
    out_ref[:, D:] = g1


def _tc_tables(cnrows3, Wc, aggW1T, aggb1):
    nb = 5
    blk = NC // nb
    return pl.pallas_call(
        _b1_body,
        grid=(nb,),
        in_specs=[pl.BlockSpec((blk, CN, D), lambda i: (i, 0, 0)),
                  pl.BlockSpec((blk, D), lambda i: (i, 0)),
                  pl.BlockSpec((D, D), lambda i: (0, 0)),
                  pl.BlockSpec((1, D), lambda i: (0, 0))],
        out_specs=pl.BlockSpec((blk, 2 * D), lambda i: (i, 0)),
        out_shape=jax.ShapeDtypeStruct((NC, 2 * D), jnp.float32),
        interpret=False,
    )(cnrows3, Wc, aggW1T, aggb1)


def _proj_body(x_ref, wt_ref, b_ref, w1_ref, qm_ref, gq_ref):
    x = x_ref[...]
    qm_ref[...] = jnp.dot(x, wt_ref[...],
                          preferred_element_type=jnp.float32) + b_ref[...]
    gq_ref[...] = jnp.dot(x, w1_ref[...], preferred_element_type=jnp.float32)


def _tc_proj(rows, WT, b, w1tile):
    n = rows.shape[0]
    blk = 2048
    nb = (n + blk - 1) // blk
    return pl.pallas_call(
        _proj_body,
        grid=(nb,),
        in_specs=[pl.BlockSpec((blk, D), lambda i: (i, 0)),
                  pl.BlockSpec((D, D), lambda i: (0, 0)),
                  pl.BlockSpec((1, D), lambda i: (0, 0)),
                  pl.BlockSpec((D, 8), lambda i: (0, 0))],
        out_specs=[pl.BlockSpec((blk, D), lambda i: (i, 0)),
                   pl.BlockSpec((blk, 8), lambda i: (i, 0))],
        out_shape=[jax.ShapeDtypeStruct((n, D), jnp.float32),
                   jax.ShapeDtypeStruct((n, 8), jnp.float32)],
        interpret=False,
    )(rows, WT, b, w1tile)


def _topk_body(e_ref, sel_ref):
    e = e_ref[0]
    sc = lax.dot_general(e, e, (((1,), (1,)), ((), ())),
                         preferred_element_type=jnp.float32)
    s = sc[1:, :]
    tcol = lax.broadcasted_iota(jnp.int32, (S - 1, S), 0)
    jcol = lax.broadcasted_iota(jnp.int32, (S - 1, S), 1)
    s = jnp.where(jcol < tcol, s, -1e30)
    sel_ref[0] = jnp.zeros((S, 16), jnp.int32)
    sels = []
    for _ in range(RK):
        m = jnp.max(s, axis=1, keepdims=True)
        idx = jnp.min(jnp.where(s >= m, jcol, S + 1), axis=1, keepdims=True)
        sels.append(idx)
        s = jnp.where(jcol == idx, -1e30, s)
    sel_ref[0, 0:S - 1, 0:RK] = jnp.concatenate(sels, axis=1)


def _tc_topk(Eb):
    return pl.pallas_call(
        _topk_body,
        grid=(B,),
        in_specs=[pl.BlockSpec((1, S, D), lambda i: (i, 0, 0))],
        out_specs=pl.BlockSpec((1, S, 16), lambda i: (i, 0, 0)),
        out_shape=jax.ShapeDtypeStruct((B, S, 16), jnp.int32),
        interpret=False,
    )(Eb)


def _d_body(tr_ref, e_ref, mc_ref, wcorr_ref, w0t_ref, b0_ref,
            wlt_ref, bl_ref, wih1t_ref, bih1_ref, gi_ref):
    sm = jnp.mean(tr_ref[...], axis=1)
    e = e_ref[...]
    f0 = jnp.tanh(jnp.dot(sm[:, :D] + e, w0t_ref[...],
                          preferred_element_type=jnp.float32) + b0_ref[...])
    f0 = jnp.tanh(jnp.dot(sm[:, D:] + f0, w0t_ref[...],
                          preferred_element_type=jnp.float32) + b0_ref[...])
    embq = jnp.tanh(jnp.dot(f0, wlt_ref[...],
                            preferred_element_type=jnp.float32) + bl_ref[...])
    mask = mc_ref[:, 0:1]
    embq = jnp.where(mask > 0.5, embq, e)
    corr = mc_ref[:, 1:2]
    embr = jnp.where(corr > 0.5, wcorr_ref[1:2, :], wcorr_ref[0:1, :])
    x = jnp.concatenate([embq, embr], axis=1)
    gi_ref[...] = jnp.dot(x, wih1t_ref[...],
                          preferred_element_type=jnp.float32) + bih1_ref[...]


def _tc_dense(Trows3, E, mc, Wcorr, W0T, b0, WLT, bl, Wih1T, bih1):
    blk = 640
    nb = (B * S) // blk
    return pl.pallas_call(
        _d_body,
        grid=(nb,),
        in_specs=[pl.BlockSpec((blk, QN, 2 * D), lambda i: (i, 0, 0)),
                  pl.BlockSpec((blk, D), lambda i: (i, 0)),
                  pl.BlockSpec((blk, 8), lambda i: (i, 0)),
                  pl.BlockSpec((2, D), lambda i: (0, 0)),
                  pl.BlockSpec((D, D), lambda i: (0, 0)),
                  pl.BlockSpec((1, D), lambda i: (0, 0)),
                  pl.BlockSpec((D, D), lambda i: (0, 0)),
                  pl.BlockSpec((1, D), lambda i: (0, 0)),
                  pl.BlockSpec((2 * D, 3 * D), lambda i: (0, 0)),
                  pl.BlockSpec((1, 3 * D), lambda i: (0, 0))],
        out_specs=pl.BlockSpec((blk, 3 * D), lambda i: (i, 0)),
        out_shape=jax.ShapeDtypeStruct((B * S, 3 * D), jnp.float32),
        interpret=False,
    )(Trows3, E, mc, Wcorr, W0T, b0, WLT, bl, Wih1T, bih1)


def _gates(gi, gh, h):
    i_r, i_z, i_n = gi[:, :D], gi[:, D:2 * D], gi[:, 2 * D:]
    h_r, h_z, h_n = gh[:, :D], gh[:, D:2 * D], gh[:, 2 * D:]
    r = jax.nn.sigmoid(i_r + h_r)
    z = jax.nn.sigmoid(i_z + h_z)
    n = jnp.tanh(i_n + r * h_n)
    return (1.0 - z) * n + z * h


def _seq_body(gi1_ref, h1i_ref, h2i_ref, whh1t_ref, bhh1_ref,
              wih2t_ref, bih2_ref, whh2t_ref, bhh2_ref,
              kwt_ref, kb_ref, w2t_ref, st_ref, kr_ref, gw_ref):
    whh1t = whh1t_ref[...]
    wih2t = wih2t_ref[...]
    whh2t = whh2t_ref[...]
    bhh1 = bhh1_ref[...]
    bih2 = bih2_ref[...]
    bhh2 = bhh2_ref[...]

    def step(t, carry):
        h1, h2 = carry
        gi = gi1_ref[pl.ds(t, 1)].reshape(B, 3 * D)
        gh1 = jnp.dot(h1, whh1t, preferred_element_type=jnp.float32) + bhh1
        h1n = _gates(gi, gh1, h1)
        gi2 = jnp.dot(h1n, wih2t, preferred_element_type=jnp.float32) + bih2
        gh2 = jnp.dot(h2, whh2t, preferred_element_type=jnp.float32) + bhh2
        out = _gates(gi2, gh2, h2)
        st_ref[pl.ds(t, 1)] = out.reshape(1, B, D)
        h2n = jnp.where(t == 0, h2, out)
        return h1n, h2n

    lax.fori_loop(0, S - 1, step, (h1i_ref[...], h2i_ref[...]))
    flat = st_ref[...].reshape((S - 1) * B, D)
    kr_ref[...] = jnp.dot(flat, kwt_ref[...],
                          preferred_element_type=jnp.float32) + kb_ref[...]
    gw_ref[...] = jnp.dot(flat, w2t_ref[...],
                          preferred_element_type=jnp.float32)


def _tc_seq(gi1_t, h1i, h2i, Whh1T, bhh1, Wih2T, bih2, Whh2T, bhh2,
            kWT, kb, w2tile):
    return pl.pallas_call(
        _seq_body,
        out_shape=[jax.ShapeDtypeStruct((S - 1, B, D), jnp.float32),
                   jax.ShapeDtypeStruct(((S - 1) * B, D), jnp.float32),
                   jax.ShapeDtypeStruct(((S - 1) * B, 8), jnp.float32)],
        interpret=False,
    )(gi1_t, h1i, h2i, Whh1T, bhh1, Wih2T, bih2, Whh2T, bhh2,
      kWT, kb, w2tile)


def _att_body(qm_ref, ss_ref, kc_ref, gq_ref, gw_ref, mk_ref,
              kwt_ref, kb_ref, w2r_ref, out_ref):
    qm = qm_ref[...]
    kc = kc_ref[...]
    l0 = jnp.sum(qm * kc[:, None, :], axis=-1)
    ss = ss_ref[...]
    ks = lax.dot_general(ss, kwt_ref[...], (((2,), (0,)), ((), ())),
                         preferred_element_type=jnp.float32)
    ks = ks + kb_ref[...][None, :, :]
    lk = jnp.sum(qm[:, :, None, :] * ks[:, None, :, :], axis=-1)
    logits = jnp.concatenate([l0[:, :, None], lk], axis=2)
    mk = mk_ref[:, :RK + 1][:, None, :]
    lm = jnp.where(mk > 0.5, logits, -1e30)
    mx = jnp.max(lm, axis=(1, 2), keepdims=True)
    ex = jnp.exp(lm - mx)
    alpha = ex / jnp.sum(ex, axis=(1, 2), keepdims=True)
    ghs = jnp.sum(ss * w2r_ref[...][:, None, :], axis=-1)
    gh = jnp.concatenate([gw_ref[:, 0:1], ghs], axis=1)
    g = jax.nn.sigmoid(gq_ref[:, :NCPQ + 1][:, :, None] +
                       gh[:, None, :])
    pred = jnp.sum(jnp.where(mk > 0.5, alpha * g, 0.0), axis=(1, 2))
    out_ref[...] = jnp.broadcast_to(pred[:, None], pred.shape + (8,))


def _tc_att(Qm, Ssel, Kc, gq, gw8, maskf, kWT, kb, w2r):
    n = (S - 1) * B
    blk = 1056
    nb = n // blk
    return pl.pallas_call(
        _att_body,
        grid=(nb,),
        in_specs=[pl.BlockSpec((blk, NCPQ + 1, D), lambda i: (i, 0, 0)),
                  pl.BlockSpec((blk, RK, D), lambda i: (i, 0, 0)),
                  pl.BlockSpec((blk, D), lambda i: (i, 0)),
                  pl.BlockSpec((blk, 8), lambda i: (i, 0)),
                  pl.BlockSpec((blk, 8), lambda i: (i, 0)),
                  pl.BlockSpec((blk, 16), lambda i: (i, 0)),
                  pl.BlockSpec((D, D), lambda i: (0, 0)),
                  pl.BlockSpec((1, D), lambda i: (0, 0)),
                  pl.BlockSpec((1, D), lambda i: (0, 0))],
        out_specs=pl.BlockSpec((blk, 8), lambda i: (i, 0)),
        out_shape=jax.ShapeDtypeStruct((n, 8), jnp.float32),
        interpret=False,
    )(Qm, Ssel, Kc, gq, gw8, maskf, kWT, kb, w2r)


def kernel(question_seq, correctness_seq, mask_seq, question_neighbors,
           concept_neighbors, q2c, Wq, Wc, Wcorr,
           gru1_Wih, gru1_Whh, gru1_bih, gru1_bhh,
           gru2_Wih, gru2_Whh, gru2_bih, gru2_bhh,
           agg_W, agg_b, agg_last_W, agg_last_b,
           q_W, q_b, k_W, k_b, w_W, w_b, h1_init, h2_init):
    f32 = jnp.float32
    qflat = question_seq.T.reshape(-1)
    qnc8 = jnp.concatenate([question_neighbors, q2c],
                           axis=1).reshape(NQ * 2 * QN // D, D)
    qrowidx = qflat // 16
    cn_pad = jnp.concatenate(
        [concept_neighbors.reshape(-1),
         jnp.zeros((480,), jnp.int32)])

    E, qncrows, cnrows = _sc_stage1(Wq, qnc8, qflat, qrowidx, cn_pad)
    off = (qflat % 16)[:, None] * (2 * QN) + jnp.arange(2 * QN)[None, :]
    ext = jnp.take_along_axis(qncrows, off, axis=1)
    n1flat = ext[:, :QN].reshape(-1)
    c4flat = ext[:, QN:].reshape(-1)

    Wc4 = _sc_gather_wc(Wc, c4flat)

    w1 = w_W[:, :D].T
    w2 = w_W[:, D:].T
    w1tile = jnp.broadcast_to(w1, (D, 8))
    w2tile = jnp.broadcast_to(w2, (D, 8))

    Tmean = _tc_tables(cnrows[:NC * CN].reshape(NC, CN, D), Wc,
                       agg_W[1].T, agg_b[1].reshape(1, D))
    Qm_c, gq_c8 = _tc_proj(Wc4, q_W.T, q_b.reshape(1, D), w1tile)
    Qm_q, gq_q8 = _tc_proj(E, q_W.T, q_b.reshape(1, D), w1tile)

    sel16 = _tc_topk(E.reshape(S, B, D).transpose(1, 0, 2))
    sel = sel16[:, :S - 1, :RK]

    Trows = _sc_gather_tmean(Tmean, n1flat)

    mc = jnp.zeros((B * S, 8), f32)
    mc = mc.at[:, 0].set((mask_seq.T.reshape(-1) != 0).astype(f32))
    mc = mc.at[:, 1].set(correctness_seq.T.reshape(-1).astype(f32))

    gi1 = _tc_dense(Trows.reshape(B * S, QN, 2 * D), E, mc, Wcorr,
                    agg_W[0].T, agg_b[0].reshape(1, D),
                    agg_last_W.T, agg_last_b.reshape(1, D),
                    gru1_Wih.T, gru1_bih.reshape(1, 3 * D))

    states, K_rows, gw8 = _tc_seq(
        gi1.reshape(S, B, 3 * D), h1_init, h2_init,
        gru1_Whh.T, gru1_bhh.reshape(1, 3 * D),
        gru2_Wih.T, gru2_bih.reshape(1, 3 * D),
        gru2_Whh.T, gru2_bhh.reshape(1, 3 * D),
        k_W.T, k_b.reshape(1, D), w2tile)

    srows = states.reshape((S - 1) * B, D)
    statetab = jnp.concatenate(
        [jnp.zeros((B, D), f32), srows[B:]], axis=0)

    bcol = jnp.arange(B, dtype=jnp.int32)[:, None, None]
    fid = (sel * B + bcol).transpose(1, 0, 2).reshape(-1)
    fid = jnp.concatenate([fid, jnp.zeros((64000 - fid.shape[0],), jnp.int32)])
    Ssel = _sc_gather_state(statetab, fid).reshape(6400, RK, D)

    tarr = jnp.arange(S - 1)
    hv = (jnp.arange(RK + 1)[None, :] <= jnp.minimum(tarr, RK)[:, None])
    maskf = jnp.zeros((S - 1, 16), f32).at[:, :RK + 1].set(hv.astype(f32))
    maskf = jnp.broadcast_to(maskf[:, None, :], (S - 1, B, 16)).reshape(-1, 16)

    Qm = jnp.concatenate(
        [Qm_q.reshape(S, B, 1, D)[1:],
         Qm_c.reshape(S, B, NCPQ, D)[1:]], axis=2).reshape(-1, NCPQ + 1, D)
    gq = jnp.concatenate(
        [gq_q8.reshape(S, B, 8)[1:, :, :1],
         gq_c8.reshape(S, B, NCPQ, 8)[1:, :, :, 0]], axis=2).reshape(-1, NCPQ + 1)
    gq = gq + w_b[0]
    gq = jnp.concatenate(
        [gq, jnp.zeros(((S - 1) * B, 8 - NCPQ - 1), f32)], axis=1)

    pred8 = _tc_att(Qm, Ssel, K_rows, gq, gw8, maskf,
                    k_W.T, k_b.reshape(1, D), w2.reshape(1, D))
    pred = pred8[:, 0].reshape(S - 1, B).T

    y = jnp.concatenate(
        [pred[:, :1], jnp.zeros((B, 1), f32), pred[:, 1:]], axis=1)
    return y


# speedup vs baseline: 10.3571x; 1.1536x over previous
import functools
import jax
import jax.numpy as jnp
from jax import lax
from jax.experimental import pallas as pl
from jax.experimental.pallas import tpu as pltpu
from jax.experimental.pallas import tpu_sc as plsc

NQ, NC, D = 50000, 2000, 128
B, S = 64, 100
QN, CN, NCPQ, RK = 4, 10, 4, 10
NW = 32


def _lazy(builder):
    box = []

    def f(*a):
        if not box:
            box.append(builder())
        return box[0](*a)

    return f


def _make_sc_gather(V, Drow, dtype, Npad, group, chunk):
    pw = Npad // NW
    ng = pw // group
    assert Npad % NW == 0 and pw % group == 0 and group % chunk == 0
    assert chunk <= 128 and chunk % 8 == 0 and pw % 8 == 0

    @functools.partial(
        pl.kernel,
        mesh=plsc.VectorSubcoreMesh(core_axis_name="c", subcore_axis_name="s"),
        out_type=jax.ShapeDtypeStruct((Npad, Drow), dtype),
        scratch_types=[
            pltpu.VMEM((pw,), jnp.int32),
            pltpu.VMEM((group, Drow), dtype),
            pltpu.VMEM((group, Drow), dtype),
            pltpu.SemaphoreType.DMA,
            pltpu.SemaphoreType.DMA,
        ],
    )
    def k(table_hbm, idx_hbm, out_hbm, idx_v, rows0, rows1, semA, semB):
        wid = lax.axis_index("s") * 2 + lax.axis_index("c")
        base = wid * pw
        pltpu.sync_copy(idx_hbm.at[pl.ds(base, pw)], idx_v)
        bufs = (rows0, rows1)
        sems = (semA, semB)

        def fire(g):
            return [
                pltpu.async_copy(
                    table_hbm.at[idx_v.at[pl.ds(g * group + j * chunk, chunk)]],
                    bufs[g % 2].at[pl.ds(j * chunk, chunk)], sems[g % 2])
                for j in range(group // chunk)
            ]

        pend = fire(0)
        for g in range(ng):
            nxt = fire(g + 1) if g + 1 < ng else []
            for c in pend:
                c.wait()
            pltpu.sync_copy(bufs[g % 2], out_hbm.at[pl.ds(base + g * group, group)])
            pend = nxt

    return k


def _build_sc_stage1():
    @functools.partial(
        pl.kernel,
        mesh=plsc.VectorSubcoreMesh(core_axis_name="c", subcore_axis_name="s"),
        out_type=[
            jax.ShapeDtypeStruct((B * S, D), jnp.float32),
            jax.ShapeDtypeStruct((B * S, D), jnp.int32),
            jax.ShapeDtypeStruct((NC * CN + 480, D), jnp.float32),
        ],
        scratch_types=[
            pltpu.VMEM((200,), jnp.int32),
            pltpu.VMEM((200,), jnp.int32),
            pltpu.VMEM((200, D), jnp.float32),
            pltpu.VMEM((200, D), jnp.int32),
            pltpu.VMEM((640,), jnp.int32),
            pltpu.VMEM((320, D), jnp.float32),
            pltpu.SemaphoreType.DMA,
            pltpu.SemaphoreType.DMA,
        ],
    )
    def k(wq_hbm, qnc_hbm, qidx_hbm, qridx_hbm, cidx_hbm,
          e_hbm, qncrows_hbm, cnrows_hbm,
          qi_v, qri_v, erows_v, qrows_v, ci_v, crows_v, sem, sem2):
        wid = lax.axis_index("s") * 2 + lax.axis_index("c")
        qb = wid * 200
        pltpu.sync_copy(qidx_hbm.at[pl.ds(qb, 200)], qi_v)
        pltpu.sync_copy(qridx_hbm.at[pl.ds(qb, 200)], qri_v)
        cps = [pltpu.async_copy(wq_hbm.at[qi_v.at[pl.ds(0, 104)]],
                                erows_v.at[pl.ds(0, 104)], sem),
               pltpu.async_copy(wq_hbm.at[qi_v.at[pl.ds(104, 96)]],
                                erows_v.at[pl.ds(104, 96)], sem),
               pltpu.async_copy(qnc_hbm.at[qri_v.at[pl.ds(0, 104)]],
                                qrows_v.at[pl.ds(0, 104)], sem),
               pltpu.async_copy(qnc_hbm.at[qri_v.at[pl.ds(104, 96)]],
                                qrows_v.at[pl.ds(104, 96)], sem)]
        cb = wid * 640
        pltpu.sync_copy(cidx_hbm.at[pl.ds(cb, 640)], ci_v)
        for g in range(2):
            cg = [pltpu.async_copy(
                wq_hbm.at[ci_v.at[pl.ds(g * 320 + off, n)]],
                crows_v.at[pl.ds(off, n)], sem2)
                for off, n in ((0, 128), (128, 128), (256, 64))]
            for c in cg:
                c.wait()
            pltpu.sync_copy(crows_v, cnrows_hbm.at[pl.ds(cb + g * 320, 320)])
        for c in cps:
            c.wait()
        pltpu.sync_copy(erows_v, e_hbm.at[pl.ds(qb, 200)])
        pltpu.sync_copy(qrows_v, qncrows_hbm.at[pl.ds(qb, 200)])

    return k


_sc_stage1 = _lazy(_build_sc_stage1)
_sc_gather_wc = _lazy(lambda: _make_sc_gather(NC, D, jnp.float32, 25600, 400, 80))
_sc_gather_tmean = _lazy(lambda: _make_sc_gather(NC, 2 * D, jnp.float32, 25600, 200, 40))
_sc_gather_state = _lazy(lambda: _make_sc_gather((S - 1) * B, D, jnp.float32, 64000, 400, 80))


def _b1_body(cn_ref, wc_ref, w1t_ref, b1_ref, out_ref):
    m = jnp.mean(cn_ref[...], axis=1)
    g1 = jnp.tanh(jnp.dot(m + wc_ref[...], w1t_ref[...],
                          preferred_element_type=jnp.float32) + b1_ref[...])
    out_ref[:, :D] = wc_ref[...]
    out_ref[:, D:] = g1


def _tc_tables(cnrows3, Wc, aggW1T, aggb1):
    nb = 5
    blk = NC // nb
    return pl.pallas_call(
        _b1_body,
        grid=(nb,),
        in_specs=[pl.BlockSpec((blk, CN, D), lambda i: (i, 0, 0)),
                  pl.BlockSpec((blk, D), lambda i: (i, 0)),
                  pl.BlockSpec((D, D), lambda i: (0, 0)),
                  pl.BlockSpec((1, D), lambda i: (0, 0))],
        out_specs=pl.BlockSpec((blk, 2 * D), lambda i: (i, 0)),
        out_shape=jax.ShapeDtypeStruct((NC, 2 * D), jnp.float32),
        interpret=False,
    )(cnrows3, Wc, aggW1T, aggb1)


def _proj_body(x_ref, wt_ref, b_ref, w1_ref, qm_ref, gq_ref):
    x = x_ref[...]
    qm_ref[...] = jnp.dot(x, wt_ref[...],
                          preferred_element_type=jnp.float32) + b_ref[...]
    gq_ref[...] = jnp.dot(x, w1_ref[...], preferred_element_type=jnp.float32)


def _tc_proj(rows, WT, b, w1tile):
    n = rows.shape[0]
    blk = 2048
    nb = (n + blk - 1) // blk
    return pl.pallas_call(
        _proj_body,
        grid=(nb,),
        in_specs=[pl.BlockSpec((blk, D), lambda i: (i, 0)),
                  pl.BlockSpec((D, D), lambda i: (0, 0)),
                  pl.BlockSpec((1, D), lambda i: (0, 0)),
                  pl.BlockSpec((D, 8), lambda i: (0, 0))],
        out_specs=[pl.BlockSpec((blk, D), lambda i: (i, 0)),
                   pl.BlockSpec((blk, 8), lambda i: (i, 0))],
        out_shape=[jax.ShapeDtypeStruct((n, D), jnp.float32),
                   jax.ShapeDtypeStruct((n, 8), jnp.float32)],
        interpret=False,
    )(rows, WT, b, w1tile)


def _topk_body(e_ref, sel_ref):
    e = e_ref[0]
    sc = lax.dot_general(e, e, (((1,), (1,)), ((), ())),
                         preferred_element_type=jnp.float32)
    s = sc[1:, :]
    tcol = lax.broadcasted_iota(jnp.int32, (S - 1, S), 0)
    jcol = lax.broadcasted_iota(jnp.int32, (S - 1, S), 1)
    s = jnp.where(jcol < tcol, s, -1e30)
    sel_ref[0] = jnp.zeros((S, 16), jnp.int32)
    sels = []
    for _ in range(RK):
        m = jnp.max(s, axis=1, keepdims=True)
        idx = jnp.min(jnp.where(s >= m, jcol, S + 1), axis=1, keepdims=True)
        sels.append(idx)
        s = jnp.where(jcol == idx, -1e30, s)
    sel_ref[0, 0:S - 1, 0:RK] = jnp.concatenate(sels, axis=1)


def _tc_topk(Eb):
    return pl.pallas_call(
        _topk_body,
        grid=(B,),
        in_specs=[pl.BlockSpec((1, S, D), lambda i: (i, 0, 0))],
        out_specs=pl.BlockSpec((1, S, 16), lambda i: (i, 0, 0)),
        out_shape=jax.ShapeDtypeStruct((B, S, 16), jnp.int32),
        interpret=False,
    )(Eb)


def _d_body(tr0_ref, tr1_ref, tr2_ref, tr3_ref, e_ref, mc_ref, wcorr_ref,
            w0t_ref, b0_ref, wlt_ref, bl_ref, wih1t_ref, bih1_ref, gi_ref):
    sm = (tr0_ref[...] + tr1_ref[...] + tr2_ref[...] + tr3_ref[...]) * 0.25
    e = e_ref[...]
    f0 = jnp.tanh(jnp.dot(sm[:, :D] + e, w0t_ref[...],
                          preferred_element_type=jnp.float32) + b0_ref[...])
    f0 = jnp.tanh(jnp.dot(sm[:, D:] + f0, w0t_ref[...],
                          preferred_element_type=jnp.float32) + b0_ref[...])
    embq = jnp.tanh(jnp.dot(f0, wlt_ref[...],
                            preferred_element_type=jnp.float32) + bl_ref[...])
    mask = mc_ref[:, 0:1]
    embq = jnp.where(mask > 0.5, embq, e)
    corr = mc_ref[:, 1:2]
    embr = jnp.where(corr > 0.5, wcorr_ref[1:2, :], wcorr_ref[0:1, :])
    x = jnp.concatenate([embq, embr], axis=1)
    gi = jnp.dot(x, wih1t_ref[...],
                 preferred_element_type=jnp.float32) + bih1_ref[...]
    gi_ref[...] = gi.reshape(gi_ref.shape)


def _tc_dense(Trows, E, mc, Wcorr, W0T, b0, WLT, bl, Wih1T, bih1):
    blk = 640
    nb = (B * S) // blk
    trspec = [pl.BlockSpec((blk, 2 * D), functools.partial(
        lambda k, i: (k * nb + i, 0), k)) for k in range(QN)]
    return pl.pallas_call(
        _d_body,
        grid=(nb,),
        in_specs=trspec +
                 [pl.BlockSpec((blk, D), lambda i: (i, 0)),
                  pl.BlockSpec((blk, 8), lambda i: (i, 0)),
                  pl.BlockSpec((2, D), lambda i: (0, 0)),
                  pl.BlockSpec((D, D), lambda i: (0, 0)),
                  pl.BlockSpec((1, D), lambda i: (0, 0)),
                  pl.BlockSpec((D, D), lambda i: (0, 0)),
                  pl.BlockSpec((1, D), lambda i: (0, 0)),
                  pl.BlockSpec((2 * D, 3 * D), lambda i: (0, 0)),
                  pl.BlockSpec((1, 3 * D), lambda i: (0, 0))],
        out_specs=pl.BlockSpec((blk // B, B, 3 * D), lambda i: (i, 0, 0)),
        out_shape=jax.ShapeDtypeStruct((S, B, 3 * D), jnp.float32),
        interpret=False,
    )(Trows, Trows, Trows, Trows, E, mc, Wcorr, W0T, b0, WLT, bl, Wih1T, bih1)


def _gates(gi, gh, h):
    i_r, i_z, i_n = gi[:, :D], gi[:, D:2 * D], gi[:, 2 * D:]
    h_r, h_z, h_n = gh[:, :D], gh[:, D:2 * D], gh[:, 2 * D:]
    r = jax.nn.sigmoid(i_r + h_r)
    z = jax.nn.sigmoid(i_z + h_z)
    n = jnp.tanh(i_n + r * h_n)
    return (1.0 - z) * n + z * h


def _seq_body(gi1_ref, h1i_ref, h2i_ref, whh1t_ref, bhh1_ref,
              wih2t_ref, bih2_ref, whh2t_ref, bhh2_ref,
              kwt_ref, kb_ref, w2t_ref, st_ref, kr_ref, gw_ref):
    whh1t = whh1t_ref[...]
    wih2t = wih2t_ref[...]
    whh2t = whh2t_ref[...]
    bhh1 = bhh1_ref[...]
    bih2 = bih2_ref[...]
    bhh2 = bhh2_ref[...]

    def step(t, carry):
        h1, h2 = carry
        gi = gi1_ref[pl.ds(t, 1)].reshape(B, 3 * D)
        gh1 = jnp.dot(h1, whh1t, preferred_element_type=jnp.float32) + bhh1
        h1n = _gates(gi, gh1, h1)
        gi2 = jnp.dot(h1n, wih2t, preferred_element_type=jnp.float32) + bih2
        gh2 = jnp.dot(h2, whh2t, preferred_element_type=jnp.float32) + bhh2
        out = _gates(gi2, gh2, h2)
        st_ref[pl.ds(t, 1)] = out.reshape(1, B, D)
        h2n = jnp.where(t == 0, h2, out)
        return h1n, h2n

    lax.fori_loop(0, S - 1, step, (h1i_ref[...], h2i_ref[...]))
    flat = st_ref[...].reshape((S - 1) * B, D)
    kr_ref[...] = jnp.dot(flat, kwt_ref[...],
                          preferred_element_type=jnp.float32) + kb_ref[...]
    gw_ref[...] = jnp.dot(flat, w2t_ref[...],
                          preferred_element_type=jnp.float32)


def _tc_seq(gi1_t, h1i, h2i, Whh1T, bhh1, Wih2T, bih2, Whh2T, bhh2,
            kWT, kb, w2tile):
    return pl.pallas_call(
        _seq_body,
        out_shape=[jax.ShapeDtypeStruct((S - 1, B, D), jnp.float32),
                   jax.ShapeDtypeStruct(((S - 1) * B, D), jnp.float32),
                   jax.ShapeDtypeStruct(((S - 1) * B, 8), jnp.float32)],
        interpret=False,
    )(gi1_t, h1i, h2i, Whh1T, bhh1, Wih2T, bih2, Whh2T, bhh2,
      kWT, kb, w2tile)


def _att_body(qm_ref, ss_ref, kc_ref, gq_ref, gw_ref, mk_ref,
              kwt_ref, kb_ref, w2r_ref, out_ref):
    blk = qm_ref.shape[0]
    qm = qm_ref[...]
    kc = kc_ref[...]
    l0 = jnp.sum(qm * kc[:, None, :], axis=-1)
    ssf = ss_ref[...]
    ks = jnp.dot(ssf, kwt_ref[...],
                 preferred_element_type=jnp.float32) + kb_ref[...]
    ks3 = ks.reshape(blk, RK, D)
    ss3 = ssf.reshape(blk, RK, D)
    gh3 = jnp.sum(ss3 * w2r_ref[...][:, None, :], axis=-1)
    pieces = [l0[:, :, None]]
    for h in range(RK):
        lh = jnp.sum(qm * ks3[:, h][:, None, :], axis=-1)
        pieces.append(lh[:, :, None])
    logits = jnp.concatenate(pieces, axis=2)
    mk = mk_ref[:, :RK + 1][:, None, :]
    lm = jnp.where(mk > 0.5, logits, -1e30)
    mx = jnp.max(lm, axis=(1, 2), keepdims=True)
    ex = jnp.exp(lm - mx)
    alpha = ex / jnp.sum(ex, axis=(1, 2), keepdims=True)
    gh = jnp.concatenate([gw_ref[:, 0:1], gh3], axis=1)
    g = jax.nn.sigmoid(gq_ref[:, :NCPQ + 1][:, :, None] +
                       gh[:, None, :])
    pred = jnp.sum(jnp.where(mk > 0.5, alpha * g, 0.0), axis=(1, 2))
    out_ref[...] = jnp.broadcast_to(pred[:, None], pred.shape + (8,))


def _tc_att(Qm, Sselraw, Kc, gq, gw8, maskf, kWT, kb, w2r):
    n = (S - 1) * B
    blk = 1056
    nb = n // blk
    return pl.pallas_call(
        _att_body,
        grid=(nb,),
        in_specs=[pl.BlockSpec((blk, NCPQ + 1, D), lambda i: (i, 0, 0)),
                  pl.BlockSpec((blk * RK, D), lambda i: (i, 0)),
                  pl.BlockSpec((blk, D), lambda i: (i, 0)),
                  pl.BlockSpec((blk, 8), lambda i: (i, 0)),
                  pl.BlockSpec((blk, 8), lambda i: (i, 0)),
                  pl.BlockSpec((blk, 16), lambda i: (i, 0)),
                  pl.BlockSpec((D, D), lambda i: (0, 0)),
                  pl.BlockSpec((1, D), lambda i: (0, 0)),
                  pl.BlockSpec((1, D), lambda i: (0, 0))],
        out_specs=pl.BlockSpec((blk, 8), lambda i: (i, 0)),
        out_shape=jax.ShapeDtypeStruct((n, 8), jnp.float32),
        interpret=False,
    )(Qm, Sselraw, Kc, gq, gw8, maskf, kWT, kb, w2r)


def kernel(question_seq, correctness_seq, mask_seq, question_neighbors,
           concept_neighbors, q2c, Wq, Wc, Wcorr,
           gru1_Wih, gru1_Whh, gru1_bih, gru1_bhh,
           gru2_Wih, gru2_Whh, gru2_bih, gru2_bhh,
           agg_W, agg_b, agg_last_W, agg_last_b,
           q_W, q_b, k_W, k_b, w_W, w_b, h1_init, h2_init):
    f32 = jnp.float32
    qflat = question_seq.T.reshape(-1)
    qnc8 = jnp.concatenate([question_neighbors, q2c],
                           axis=1).reshape(NQ * 2 * QN // D, D)
    qrowidx = qflat // 16
    cn_pad = jnp.concatenate(
        [concept_neighbors.reshape(-1),
         jnp.zeros((480,), jnp.int32)])

    E, qncrows, cnrows = _sc_stage1(Wq, qnc8, qflat, qrowidx, cn_pad)
    off = (qflat % 16)[:, None] * (2 * QN) + jnp.arange(2 * QN)[None, :]
    ext = jnp.take_along_axis(qncrows, off, axis=1)
    n1flat = ext[:, :QN].T.reshape(-1)
    c4flat = ext[:, QN:].reshape(-1)

    Wc4 = _sc_gather_wc(Wc, c4flat)

    w1 = w_W[:, :D].T
    w2 = w_W[:, D:].T
    w1tile = jnp.broadcast_to(w1, (D, 8))
    w2tile = jnp.broadcast_to(w2, (D, 8))

    Tmean = _tc_tables(cnrows[:NC * CN].reshape(NC, CN, D), Wc,
                       agg_W[1].T, agg_b[1].reshape(1, D))
    Qm_c, gq_c8 = _tc_proj(Wc4, q_W.T, q_b.reshape(1, D), w1tile)
    Qm_q, gq_q8 = _tc_proj(E, q_W.T, q_b.reshape(1, D), w1tile)

    sel16 = _tc_topk(E.reshape(S, B, D).transpose(1, 0, 2))
    sel = sel16[:, :S - 1, :RK]

    Trows = _sc_gather_tmean(Tmean, n1flat)

    mc = jnp.concatenate(
        [(mask_seq.T.reshape(-1, 1) != 0).astype(f32),
         correctness_seq.T.reshape(-1, 1).astype(f32),
         jnp.zeros((B * S, 6), f32)], axis=1)

    gi1 = _tc_dense(Trows, E, mc, Wcorr,
                    agg_W[0].T, agg_b[0].reshape(1, D),
                    agg_last_W.T, agg_last_b.reshape(1, D),
                    gru1_Wih.T, gru1_bih.reshape(1, 3 * D))

    states, K_rows, gw8 = _tc_seq(
        gi1, h1_init, h2_init,
        gru1_Whh.T, gru1_bhh.reshape(1, 3 * D),
        gru2_Wih.T, gru2_bih.reshape(1, 3 * D),
        gru2_Whh.T, gru2_bhh.reshape(1, 3 * D),
        k_W.T, k_b.reshape(1, D), w2tile)

    srows = states.reshape((S - 1) * B, D)
    statetab = jnp.concatenate(
        [jnp.zeros((B, D), f32), srows[B:]], axis=0)

    bcol = jnp.arange(B, dtype=jnp.int32)[:, None, None]
    fid = (sel * B + bcol).transpose(1, 0, 2).reshape(-1)
    fid = jnp.concatenate([fid, jnp.zeros((64000 - fid.shape[0],), jnp.int32)])
    Ssel = _sc_gather_state(statetab, fid)

    tarr = jnp.arange(S - 1)
    hv = (jnp.arange(RK + 1)[None, :] <= jnp.minimum(tarr, RK)[:, None])
    maskf = jnp.zeros((S - 1, 16), f32).at[:, :RK + 1].set(hv.astype(f32))
    maskf = jnp.broadcast_to(maskf[:, None, :], (S - 1, B, 16)).reshape(-1, 16)

    Qm = jnp.concatenate(
        [Qm_q.reshape(S, B, 1, D)[1:],
         Qm_c.reshape(S, B, NCPQ, D)[1:]], axis=2).reshape(-1, NCPQ + 1, D)
    gq = jnp.concatenate(
        [gq_q8.reshape(S, B, 8)[1:, :, :1],
         gq_c8.reshape(S, B, NCPQ, 8)[1:, :, :, 0]], axis=2).reshape(-1, NCPQ + 1)
    gq = gq + w_b[0]
    gq = jnp.concatenate(
        [gq, jnp.zeros(((S - 1) * B, 8 - NCPQ - 1), f32)], axis=1)

    pred8 = _tc_att(Qm, Ssel, K_rows, gq, gw8, maskf,
                    k_W.T, k_b.reshape(1, D), w2.reshape(1, D))
    pred = pred8[:, 0].reshape(S - 1, B).T

    y = jnp.concatenate(
        [pred[:, :1], jnp.zeros((B, 1), f32), pred[:, 1:]], axis=1)
    return y
